# baseline (speedup 1.0000x reference)
import functools

import jax
import jax.numpy as jnp
from jax import lax
from jax.experimental import pallas as pl
from jax.experimental.pallas import tpu as pltpu

_LRELU = 0.01


def _pad2(a, rows, cols):
  r, c = a.shape
  if r == rows and c == cols:
    return a
  return jnp.pad(a, ((0, rows - r), (0, cols - c)))


def _mm_body(a_ref, b_ref, o_ref):
  o_ref[...] = jnp.dot(a_ref[...], b_ref[...],
                       preferred_element_type=jnp.float32)


@functools.partial(jax.jit, static_argnames=("bm",))
def _mm(a, b, bm):
  m, k = a.shape
  n = b.shape[1]
  return pl.pallas_call(
      _mm_body,
      grid=(m // bm,),
      in_specs=[
          pl.BlockSpec((bm, k), lambda i: (i, 0)),
          pl.BlockSpec((k, n), lambda i: (0, 0)),
      ],
      out_specs=pl.BlockSpec((bm, n), lambda i: (i, 0)),
      out_shape=jax.ShapeDtypeStruct((m, n), jnp.float32),
  )(a, b)


def _gcn_hw_body(x_ref, z_ref, dis_ref, w1t_ref, hwt_ref, b_ref, o_ref):
  d = dis_ref[:, 0:1]
  agg = jnp.maximum(d * z_ref[...], 0.0)
  t = jnp.dot(agg, w1t_ref[...], preferred_element_type=jnp.float32)
  g = jax.nn.sigmoid(
      jnp.dot(x_ref[...], hwt_ref[...], preferred_element_type=jnp.float32)
      + b_ref[0:1, :])
  o_ref[...] = g * t + (1.0 - g) * x_ref[...]


@jax.jit
def _gcn_hw(xp, zp, disb, w1t, hwt, bp):
  m, k = xp.shape
  bm = 400
  return pl.pallas_call(
      _gcn_hw_body,
      grid=(m // bm,),
      in_specs=[
          pl.BlockSpec((bm, k), lambda i: (i, 0)),
          pl.BlockSpec((bm, k), lambda i: (i, 0)),
          pl.BlockSpec((bm, 128), lambda i: (i, 0)),
          pl.BlockSpec((k, k), lambda i: (0, 0)),
          pl.BlockSpec((k, k), lambda i: (0, 0)),
          pl.BlockSpec((8, k), lambda i: (0, 0)),
      ],
      out_specs=pl.BlockSpec((bm, k), lambda i: (i, 0)),
      out_shape=jax.ShapeDtypeStruct((m, k), jnp.float32),
  )(xp, zp, disb, w1t, hwt, bp)


def _segsum(v, idx, n):
  return jax.ops.segment_sum(v, idx, num_segments=n)


def kernel(x_e, edge_index, rel, edge_index_all, rel_all,
           line_graph_index_out, line_graph_val_out,
           line_graph_index_in, line_graph_val_in,
           rel_emb1, rel_emb2, gcn1_w, gcn2_w,
           hw1_w, hw1_b, hw2_w, hw2_b, ww1_w,
           gat_ai, gat_aj, gat_ar, gatr_ai, gatr_aj):
  n = x_e.shape[0]
  eh = x_e.shape[1]
  rh = rel_emb1.shape[1]
  nrel = rel_emb1.shape[0]
  ehp = 384
  rhp = 128

  j_all = edge_index_all[0]
  i_all = edge_index_all[1]

  deg = _segsum(jnp.ones_like(i_all, jnp.float32), i_all, n)
  dis = deg ** -0.5
  disb = jnp.broadcast_to(dis[:, None], (n, 128))

  x = x_e
  for w, hw_w, hw_b in ((gcn1_w, hw1_w, hw1_b), (gcn2_w, hw2_w, hw2_b)):
    y = dis[:, None] * x
    z = _segsum(y[j_all], i_all, n)
    xp = _pad2(x, n, ehp)
    zp = _pad2(z, n, ehp)
    w1t = _pad2(w.T, ehp, ehp)
    hwt = _pad2(hw_w.T, ehp, ehp)
    bp = _pad2(hw_b[None, :], 8, ehp)
    x = _gcn_hw(xp, zp, disb, w1t, hwt, bp)[:, :eh]

  re = jnp.where(rel.max() + 1 == nrel, rel_emb1, rel_emb2)
  rep = _pad2(re, nrel, rhp)
  vs = _pad2(jnp.stack([gatr_ai, gatr_aj], 1), rhp, 128)
  sc = _mm(rep, vs, 200)
  ai_s, aj_s = sc[:, 0], sc[:, 1]

  def gat_r(edge):
    jj, ii = edge[0], edge[1]
    e = jax.nn.leaky_relu(ai_s[ii] + aj_s[jj], _LRELU)
    mprime = jax.nn.leaky_relu(aj_s + ai_s.max(), _LRELU)
    ex = jnp.exp(e - mprime[jj])
    s = _segsum(ex, jj, nrel)
    alpha = ex / (s[jj] + 1e-16)
    return jnp.maximum(_segsum(alpha[:, None] * re[jj], ii, nrel), 0.0)

  rel_emb = jnp.concatenate(
      [gat_r(line_graph_index_out), gat_r(line_graph_index_in)], 0)

  ef = jax.nn.leaky_relu(x, _LRELU)
  rl = jax.nn.leaky_relu(rel_emb, _LRELU)
  efp = _pad2(ef, n, ehp)
  wa = _pad2(jnp.stack([ww1_w[:eh], ww1_w[eh + rh:]], 1), ehp, 128)
  sc2 = _mm(efp, wa, 400)
  a_n, c_n = sc2[:, 0], sc2[:, 1]
  rlp = _pad2(rl, 2 * nrel, rhp)
  b_r = _mm(rlp, _pad2(ww1_w[eh:eh + rh, None], rhp, 128), 400)[:, 0]

  ig, jg = edge_index_all[0], edge_index_all[1]
  e = a_n[ig] + b_r[rel_all] + c_n[jg]
  mprime = a_n + (b_r.max() + c_n.max())
  ex = jnp.exp(e - mprime[ig])
  s = _segsum(ex, ig, n)
  att = ex / (s[ig] + 1e-16)
  out1 = ef * _segsum(att, ig, n)[:, None]
  out2 = _segsum(att[:, None] * rl[rel_all], ig, n)
  out3 = _segsum(att[:, None] * ef[jg], ig, n)
  x_wjq = jnp.concatenate([x, out1, out2, out3], 1)

  dwp = 1024
  xwp = _pad2(x_wjq, n, dwp)
  va = _pad2(jnp.stack([gat_ai, gat_aj], 1), dwp, 128)
  sc3 = _mm(xwp, va, 400)
  si, sj = sc3[:, 0], sc3[:, 1]
  sr = _mm(_pad2(rel_emb, 2 * nrel, rhp),
           _pad2(gat_ar[:, None], rhp, 128), 400)[:, 0]

  e2 = si[i_all] + sj[j_all] + sr[rel_all]
  mask = j_all != i_all
  em = jnp.where(mask, jax.nn.leaky_relu(e2, _LRELU), -jnp.inf)
  mprime2 = jax.nn.leaky_relu(si + (sj.max() + sr.max()), _LRELU)
  ex2 = jnp.where(mask, jnp.exp(em - mprime2[i_all]), 0.0)
  s2 = _segsum(ex2, i_all, n)
  alpha2 = ex2 / (s2[i_all] + 1e-16)
  gat_out = jnp.maximum(_segsum(alpha2[:, None] * x_wjq[j_all], i_all, n), 0.0)

  return jnp.concatenate([x_wjq, gat_out], 1)


# speedup vs baseline: 2.5191x; 2.5191x over previous
import functools

import jax
import jax.numpy as jnp
from jax import lax
from jax.experimental import pallas as pl
from jax.experimental.pallas import tpu as pltpu
from jax.experimental.pallas import tpu_sc as plsc

_LRELU = 0.01
_NT = 32


def _mm_body(a_ref, b_ref, o_ref):
  o_ref[...] = jnp.dot(a_ref[...], b_ref[...],
                       preferred_element_type=jnp.float32)


@functools.partial(jax.jit, static_argnames=("bm",))
def _mm(a, b, bm):
  m, k = a.shape
  n = b.shape[1]
  return pl.pallas_call(
      _mm_body,
      grid=(m // bm,),
      in_specs=[
          pl.BlockSpec((bm, k), lambda i: (i, 0)),
          pl.BlockSpec((k, n), lambda i: (0, 0)),
      ],
      out_specs=pl.BlockSpec((bm, n), lambda i: (i, 0)),
      out_shape=jax.ShapeDtypeStruct((m, n), jnp.float32),
  )(a, b)


def _gcn_hw_body(x_ref, z_ref, dis_ref, w1t_ref, hwt_ref, b_ref, o_ref):
  d = dis_ref[:, 0:1]
  agg = jnp.maximum(d * z_ref[...], 0.0)
  t = jnp.dot(agg, w1t_ref[...], preferred_element_type=jnp.float32)
  g = jax.nn.sigmoid(
      jnp.dot(x_ref[...], hwt_ref[...], preferred_element_type=jnp.float32)
      + b_ref[0:1, :])
  o_ref[...] = g * t + (1.0 - g) * x_ref[...]


@jax.jit
def _gcn_hw(xp, zp, disb, w1t, hwt, bp):
  m, k = xp.shape
  bm = 400
  return pl.pallas_call(
      _gcn_hw_body,
      grid=(m // bm,),
      in_specs=[
          pl.BlockSpec((bm, k), lambda i: (i, 0)),
          pl.BlockSpec((bm, k), lambda i: (i, 0)),
          pl.BlockSpec((bm, 128), lambda i: (i, 0)),
          pl.BlockSpec((k, k), lambda i: (0, 0)),
          pl.BlockSpec((k, k), lambda i: (0, 0)),
          pl.BlockSpec((8, k), lambda i: (0, 0)),
      ],
      out_specs=pl.BlockSpec((bm, k), lambda i: (i, 0)),
      out_shape=jax.ShapeDtypeStruct((m, k), jnp.float32),
  )(xp, zp, disb, w1t, hwt, bp)


@functools.lru_cache(maxsize=None)
def _sc_softmax_sum(e_pad, e_real, nseg_pad, noth_pad, nrel_pad,
                    has_rel, use_lrelu, masked):
  ct = e_pad // (_NT * 128)
  mesh = plsc.VectorSubcoreMesh(core_axis_name="c", subcore_axis_name="s")

  scratch = [
      pltpu.VMEM((ct, 128), jnp.int32),
      pltpu.VMEM((ct, 128), jnp.int32),
      pltpu.VMEM((ct, 128), jnp.int32),
      pltpu.VMEM((nseg_pad,), jnp.float32),
      pltpu.VMEM((noth_pad,), jnp.float32),
      pltpu.VMEM((nrel_pad,), jnp.float32),
      pltpu.VMEM((nseg_pad,), jnp.float32),
      pltpu.VMEM((ct * 128,), jnp.float32),
      pltpu.VMEM((nseg_pad,), jnp.float32),
  ]

  def body(seg_h, oth_h, rel_h, a_h, b_h, c_h, mp_h, s_out, ex_out,
           seg_v, oth_v, rel_v, a_v, b_v, c_v, mp_v, exs, acc_v):
    cid = lax.axis_index("c")
    sid = lax.axis_index("s")
    wid = sid * 2 + cid
    pltpu.sync_copy(seg_h.at[pl.ds(wid * ct, ct)], seg_v)
    pltpu.sync_copy(oth_h.at[pl.ds(wid * ct, ct)], oth_v)
    if has_rel:
      pltpu.sync_copy(rel_h.at[pl.ds(wid * ct, ct)], rel_v)
    pltpu.sync_copy(a_h, a_v)
    pltpu.sync_copy(b_h, b_v)
    if has_rel:
      pltpu.sync_copy(c_h, c_v)
    pltpu.sync_copy(mp_h, mp_v)

    z16 = jnp.zeros((16,), jnp.float32)

    def zloop(i, _):
      acc_v[pl.ds(i * 16, 16)] = z16
      return 0
    lax.fori_loop(0, nseg_pad // 16, zloop, 0)

    iot = lax.iota(jnp.int32, 16)
    base_e = wid * ct * 128

    def chunk(c, _):
      def grp(g, _):
        off = pl.ds(g * 16, 16)
        si = seg_v[c, off]
        oi = oth_v[c, off]
        e = plsc.load_gather(a_v, [si]) + plsc.load_gather(b_v, [oi])
        if has_rel:
          e = e + plsc.load_gather(c_v, [rel_v[c, off]])
        if use_lrelu:
          e = jnp.where(e >= 0.0, e, e * _LRELU)
        m = plsc.load_gather(mp_v, [si])
        gid = base_e + c * 128 + g * 16 + iot
        valid = gid < e_real
        if masked:
          valid = valid & (si != oi)
        ex = jnp.where(valid, jnp.exp(e - m), 0.0)
        exs[pl.ds(c * 128 + g * 16, 16)] = ex
        for l in range(16):
          gl = plsc.load_gather(acc_v, [si])
          plsc.store_scatter(acc_v, [si], gl + ex, mask=iot == l)
        return 0
      lax.fori_loop(0, 8, grp, 0)
      return 0
    lax.fori_loop(0, ct, chunk, 0)

    pltpu.sync_copy(acc_v, s_out.at[wid])
    pltpu.sync_copy(exs, ex_out.at[pl.ds(base_e, ct * 128)])

  return pl.kernel(
      body, mesh=mesh,
      out_type=[jax.ShapeDtypeStruct((_NT, nseg_pad), jnp.float32),
                jax.ShapeDtypeStruct((e_pad,), jnp.float32)],
      scratch_types=scratch,
      compiler_params=pltpu.CompilerParams(needs_layout_passes=False),
  )


@functools.lru_cache(maxsize=None)
def _sc_rows(e_pad, n_acc, tab_rows, n_slabs, weighted, nseg_pad=0):
  ct = e_pad // (_NT * 128)
  rz = n_acc // 16
  mesh = plsc.VectorSubcoreMesh(core_axis_name="c", subcore_axis_name="s")

  scratch = [
      pltpu.VMEM((ct, 128), jnp.int32),
      pltpu.VMEM((ct, 128), jnp.int32),
      pltpu.VMEM((128, 128), jnp.float32),
      pltpu.VMEM((ct, 128), jnp.int32),
      pltpu.VMEM((ct * 128,), jnp.float32),
      pltpu.VMEM((max(nseg_pad, 16),), jnp.float32),
      pltpu.VMEM_SHARED((n_acc, 128), jnp.float32),
      pltpu.SemaphoreType.DMA,
  ]

  def body(*refs):
    pos = 0
    scat_h = refs[pos]; pos += 1
    if weighted:
      wseg_h = refs[pos]; ex_h = refs[pos + 1]; s_h = refs[pos + 2]
      pos += 3
    gidx_hs = []
    tab_hs = []
    for _ in range(n_slabs):
      gidx_hs.append(refs[pos]); tab_hs.append(refs[pos + 1]); pos += 2
    out_h = refs[pos]; pos += 1
    (scat_v, gidx_v, rows, wseg_v, ex_v, s_v, acc, sem) = refs[pos:]

    cid = lax.axis_index("c")
    sid = lax.axis_index("s")
    wid = sid * 2 + cid
    base_e = wid * ct * 128

    pltpu.sync_copy(scat_h.at[pl.ds(wid * ct, ct)], scat_v)
    if weighted:
      pltpu.sync_copy(wseg_h.at[pl.ds(wid * ct, ct)], wseg_v)
      pltpu.sync_copy(ex_h.at[pl.ds(base_e, ct * 128)], ex_v)
      pltpu.sync_copy(s_h, s_v.at[pl.ds(0, nseg_pad)])

    zrow = jnp.zeros((16,), jnp.float32)

    for t in range(n_slabs):
      pltpu.sync_copy(gidx_hs[t].at[pl.ds(wid * ct, ct)], gidx_v)

      def zloop(i, _):
        for cc in range(8):
          rows[i, pl.ds(cc * 16, 16)] = zrow
        return 0
      lax.fori_loop(0, min(rz, 128), zloop, 0)
      nzc = (rz + 127) // 128
      for q in range(nzc):
        r0 = min(q * 128, rz - min(rz, 128))
        nr = min(128, rz)
        pltpu.sync_copy(rows.at[pl.ds(0, nr)],
                        acc.at[pl.ds(sid * rz + r0, nr)])
      plsc.subcore_barrier()

      def chunk(c, _):
        pltpu.async_copy(tab_hs[t].at[gidx_v.at[c]], rows, sem).wait()
        if weighted:
          def grp(g, _):
            w16 = wseg_v[c, pl.ds(g * 16, 16)]
            ex16 = ex_v[pl.ds(c * 128 + g * 16, 16)]
            sg = plsc.load_gather(s_v, [w16])
            al = ex16 / (sg + 1e-16)
            for r in range(16):
              av = jnp.broadcast_to(al[r], (16,))
              row = g * 16 + r
              for cc in range(8):
                sl = pl.ds(cc * 16, 16)
                rows[row, sl] = rows[row, sl] * av
            return 0
          lax.fori_loop(0, 8, grp, 0)
        pltpu.sync_copy(rows, acc.at[scat_v.at[c]], add=True)
        return 0
      lax.fori_loop(0, ct, chunk, 0)

      plsc.subcore_barrier()
      pltpu.sync_copy(acc.at[pl.ds(sid * rz, rz)],
                      out_h.at[t, cid, pl.ds(sid * rz, rz)])
      if t + 1 < n_slabs:
        plsc.subcore_barrier()

  return pl.kernel(
      body, mesh=mesh,
      out_type=jax.ShapeDtypeStruct((n_slabs, 2, n_acc, 128), jnp.float32),
      scratch_types=scratch,
      compiler_params=pltpu.CompilerParams(needs_layout_passes=False),
  )


def _pad2(a, rows, cols):
  r, c = a.shape
  if r == rows and c == cols:
    return a
  return jnp.pad(a, ((0, rows - r), (0, cols - c)))


def _pad1(a, nn, val=0):
  return jnp.pad(a, (0, nn - a.shape[0]), constant_values=val)


def _to2d(idx, e_pad, pad_val):
  return _pad1(idx.astype(jnp.int32), e_pad, pad_val).reshape(e_pad // 128, 128)


def _slabs(tabp, width):
  return [tabp[:, t * 128:(t + 1) * 128] for t in range(width // 128)]


def kernel(x_e, edge_index, rel, edge_index_all, rel_all,
           line_graph_index_out, line_graph_val_out,
           line_graph_index_in, line_graph_val_in,
           rel_emb1, rel_emb2, gcn1_w, gcn2_w,
           hw1_w, hw1_b, hw2_w, hw2_b, ww1_w,
           gat_ai, gat_aj, gat_ar, gatr_ai, gatr_aj):
  n = x_e.shape[0]
  eh = x_e.shape[1]
  rh = rel_emb1.shape[1]
  nrel = rel_emb1.shape[0]
  e = edge_index_all.shape[1]
  elg = line_graph_index_out.shape[1]
  npad = 10240
  ep = 163840
  eplg = 65536
  nrp = 1024
  nr2p = 2048
  ehp = 384

  j_all = edge_index_all[0]
  i_all = edge_index_all[1]
  seg_i2 = _to2d(i_all, ep, 0)
  oth_j2 = _to2d(j_all, ep, 0)
  rel2 = _to2d(rel_all, ep, 0)
  gj2 = _to2d(j_all, ep, n)
  gi2 = _to2d(i_all, ep, n)
  grel2 = _to2d(rel_all, ep, 2 * nrel)
  zn = jnp.zeros((npad,), jnp.float32)

  cnt, _ = _sc_softmax_sum(ep, e, npad, npad, 16, False, False, False)(
      seg_i2, oth_j2, rel2, zn, zn, jnp.zeros((16,), jnp.float32), zn)
  deg = cnt[:, :n].sum(0)
  dis = deg ** -0.5
  disb = jnp.broadcast_to(dis[:, None], (n, 128))

  x = x_e
  for w, hw_w, hw_b in ((gcn1_w, hw1_w, hw1_b), (gcn2_w, hw2_w, hw2_b)):
    xp = _pad2(x, n, ehp)
    y = _pad2(dis[:, None] * x, npad, ehp)
    zz = _sc_rows(ep, npad, (npad,) * 3, 3, False)(
        seg_i2, gj2, _slabs(y, ehp)[0], gj2, _slabs(y, ehp)[1],
        gj2, _slabs(y, ehp)[2])
    z = jnp.concatenate([zz[t, 0] + zz[t, 1] for t in range(3)], 1)[:n]
    x = _gcn_hw(xp, z, disb, _pad2(w.T, ehp, ehp), _pad2(hw_w.T, ehp, ehp),
                _pad2(hw_b[None, :], 8, ehp))[:, :eh]

  re = jnp.where(rel.max() + 1 == nrel, rel_emb1, rel_emb2)
  rep = _pad2(re, nrp, 128)
  vs = _pad2(jnp.stack([gatr_ai, gatr_aj], 1), 128, 128)
  sc = _mm(rep, vs, 128)
  ai_s, aj_s = sc[:, 0], sc[:, 1]
  mp_r = jnp.where(aj_s + ai_s[:nrel].max() >= 0,
                   aj_s + ai_s[:nrel].max(),
                   (aj_s + ai_s[:nrel].max()) * _LRELU)

  def gat_r(edge):
    jj2s = _to2d(edge[0], eplg, 0)
    ii2 = _to2d(edge[1], eplg, 0)
    jj2g = _to2d(edge[0], eplg, nrel)
    s_p, ex = _sc_softmax_sum(eplg, elg, nrp, nrp, 16, False, True, False)(
        jj2s, ii2, ii2, aj_s, ai_s, jnp.zeros((16,), jnp.float32), mp_r)
    s = s_p.sum(0)
    o = _sc_rows(eplg, nrp, (nrp,), 1, True, nrp)(
        ii2, jj2s, ex, s, jj2g, rep)
    return jnp.maximum(o[0, 0] + o[0, 1], 0.0)[:nrel, :rh]

  rel_emb = jnp.concatenate(
      [gat_r(line_graph_index_out), gat_r(line_graph_index_in)], 0)

  ef = jax.nn.leaky_relu(x, _LRELU)
  rl = jax.nn.leaky_relu(rel_emb, _LRELU)
  efp = _pad2(ef, npad, ehp)
  wa = _pad2(jnp.stack([ww1_w[:eh], ww1_w[eh + rh:]], 1), ehp, 128)
  sc2 = _mm(efp, wa, 128)
  a_n, c_n = sc2[:n, 0], sc2[:n, 1]
  rlp = _pad2(rl, nr2p, 128)
  b_r = _mm(rlp, _pad2(ww1_w[eh:eh + rh, None], 128, 128), 128)[:2 * nrel, 0]

  a_np = _pad1(a_n, npad)
  mp_g = a_np + (b_r.max() + c_n.max())
  s_p, ex_g = _sc_softmax_sum(ep, e, npad, npad, nr2p, True, False, False)(
      oth_j2, seg_i2, grel2, a_np, _pad1(c_n, npad), _pad1(b_r, nr2p), mp_g)
  s_g = s_p.sum(0)
  og = _sc_rows(ep, npad, (nr2p, npad, npad, npad), 4, True, npad)(
      oth_j2, oth_j2, ex_g, s_g,
      grel2, rlp, gi2, _slabs(efp, ehp)[0], gi2, _slabs(efp, ehp)[1],
      gi2, _slabs(efp, ehp)[2])
  fac = (s_g / (s_g + 1e-16))[:n]
  out1 = ef * fac[:, None]
  out2 = (og[0, 0] + og[0, 1])[:n, :rh]
  out3 = jnp.concatenate([og[t, 0] + og[t, 1] for t in range(1, 4)], 1)[:n, :eh]
  x_wjq = jnp.concatenate([x, out1, out2, out3], 1)

  dwp = 1024
  xwp = _pad2(x_wjq, npad, dwp)
  va = _pad2(jnp.stack([gat_ai, gat_aj], 1), dwp, 128)
  sc3 = _mm(xwp[:n], va, 400)
  si, sj = sc3[:, 0], sc3[:, 1]
  sr = _mm(_pad2(rel_emb, nr2p, 128),
           _pad2(gat_ar[:, None], 128, 128), 128)[:2 * nrel, 0]

  sip = _pad1(si, npad)
  pre = sip + (sj.max() + sr.max())
  mp_t = jnp.where(pre >= 0, pre, pre * _LRELU)
  s_p2, ex_t = _sc_softmax_sum(ep, e, npad, npad, nr2p, True, True, True)(
      seg_i2, oth_j2, grel2, sip, _pad1(sj, npad), _pad1(sr, nr2p), mp_t)
  s_t = s_p2.sum(0)
  xslabs = _slabs(xwp, dwp)
  args = []
  for t in range(8):
    args += [gj2, xslabs[t]]
  ot = _sc_rows(ep, npad, (npad,) * 8, 8, True, npad)(
      seg_i2, seg_i2, ex_t, s_t, *args)
  gat_out = jnp.concatenate(
      [jnp.maximum(ot[t, 0] + ot[t, 1], 0.0) for t in range(8)], 1)[:n, :1000]

  return jnp.concatenate([x_wjq, gat_out], 1)
